# baseline (speedup 1.0000x reference)
import functools

import jax
import jax.numpy as jnp
from jax import lax
from jax.experimental import pallas as pl
from jax.experimental.pallas import tpu as pltpu
from jax.experimental.pallas import tpu_sc as plsc

_G = 128
_N = 100000
_C = 64
_NW = 32
_BLK = 128
_NBLK = 25
_CHUNK = _BLK * _NBLK
_NP = _NW * _CHUNK
_SENT = _G * _G * _G
_ZP = _N

_OFFS = [(0, d // 3 - 1, d % 3 - 1) for d in range(9)] + \
        [(d // 3 - 1, 0, d % 3 - 1) for d in range(9)]

_MESH = dict(core_axis_name="c", subcore_axis_name="s", num_cores=2,
             num_subcores=16)
_SC_PARAMS = pltpu.CompilerParams(use_tc_tiling_on_sc=False)


def _wid():
    return lax.axis_index("s") * 2 + lax.axis_index("c")


def _idx_body(cz, cy, cx, gridp, pidx, czv, cyv, cxv, linv, gv, outv, sem):
    base = _wid() * _CHUNK
    pltpu.sync_copy(cz.at[pl.ds(base, _CHUNK)], czv)
    pltpu.sync_copy(cy.at[pl.ds(base, _CHUNK)], cyv)
    pltpu.sync_copy(cx.at[pl.ds(base, _CHUNK)], cxv)

    def blk(b, carry):
        b0 = b * _BLK

        def vec(v, c2):
            s = b0 + v * 16
            z = czv[pl.ds(s, 16)]
            y = cyv[pl.ds(s, 16)]
            x = cxv[pl.ds(s, 16)]
            for t, (dz, dy, dx) in enumerate(_OFFS):
                zz = z + dz
                yy = y + dy
                xx = x + dx
                ok = (zz >= 0) & (zz < _G) & (yy >= 0) & (yy < _G) \
                    & (xx >= 0) & (xx < _G)
                lin = (zz * _G + yy) * _G + xx
                linv[t, pl.ds(v * 16, 16)] = jnp.where(ok, lin, _SENT)
            return c2

        lax.fori_loop(0, _BLK // 16, vec, 0)
        descs = [pltpu.async_copy(gridp.at[linv.at[t]], gv.at[t], sem)
                 for t in range(18)]
        for d in descs:
            d.wait()

        def vec2(v, c2):
            sl = pl.ds(v * 16, 16)
            for t in range(18):
                g = gv[t, sl]
                outv[t, sl] = jnp.where(g >= 0, g, _ZP)
            return c2

        lax.fori_loop(0, _BLK // 16, vec2, 0)
        pltpu.sync_copy(outv, pidx.at[:, pl.ds(base + b0, _BLK)])
        return carry

    lax.fori_loop(0, _NBLK, blk, 0)


def _idx_call(cz, cy, cx, gridp):
    return pl.kernel(
        _idx_body,
        out_type=jax.ShapeDtypeStruct((18, _NP), jnp.int32),
        mesh=plsc.VectorSubcoreMesh(**_MESH),
        compiler_params=_SC_PARAMS,
        scratch_types=[
            pltpu.VMEM((_CHUNK,), jnp.int32),
            pltpu.VMEM((_CHUNK,), jnp.int32),
            pltpu.VMEM((_CHUNK,), jnp.int32),
            pltpu.VMEM((18, _BLK), jnp.int32),
            pltpu.VMEM((18, _BLK), jnp.int32),
            pltpu.VMEM((18, _BLK), jnp.int32),
            pltpu.SemaphoreType.DMA,
        ],
    )(cz, cy, cx, gridp)


_SB = 320
_NSB = _CHUNK // _SB


def _make_gsum(mult, terms0, terms1):
    allt = list(terms0) + list(terms1)

    def body(table, pidx, out,
             pv0, pv1, si0, si1, a00, a01, a10, a11,
             semp0, semp1, semg0, semg1):
        base = _wid() * _CHUNK
        zero16 = jnp.zeros((16,), jnp.float32)
        pv = (pv0, pv1)
        si = (si0, si1)
        acc = ((a00, a01), (a10, a11))
        semp = (semp0, semp1)
        semg = (semg0, semg1)
        gdesc = [None] * _NSB

        def pvload(b):
            return pltpu.async_copy(
                pidx.at[:, pl.ds(base + b * _SB, _SB)], pv[b % 2],
                semp[b % 2])

        def drain_and_flush(b):
            for d in gdesc[b]:
                d.wait()
            p = b % 2
            col = base + b * _SB
            pltpu.sync_copy(acc[p][0], out.at[0, pl.ds(col, _SB)])
            pltpu.sync_copy(acc[p][1], out.at[1, pl.ds(col, _SB)])

        pd = pvload(0)
        for b in range(_NSB):
            p = b % 2
            if b >= 2:
                drain_and_flush(b - 2)
            pd.wait()
            if b + 1 < _NSB:
                pd = pvload(b + 1)

            def vec(v, c2, p=p):
                sl = pl.ds(v * 16, 16)
                for t, (row, addc) in enumerate(allt):
                    si[p][t, sl] = pv[p][row, sl] * mult + addc
                return c2

            lax.fori_loop(0, _SB // 16, vec, 0)

            def zrow(r, c2, p=p):
                for cc in range(_C // 16):
                    acc[p][0][r, pl.ds(cc * 16, 16)] = zero16
                    acc[p][1][r, pl.ds(cc * 16, 16)] = zero16
                return c2

            lax.fori_loop(0, _SB, zrow, 0)
            gdesc[b] = [
                pltpu.async_copy(table.at[si[p].at[t]],
                                 acc[p][0] if t < 9 else acc[p][1],
                                 semg[p], add=True)
                for t in range(18)
            ]
        drain_and_flush(_NSB - 2)
        drain_and_flush(_NSB - 1)

    return pl.kernel(
        body,
        out_type=jax.ShapeDtypeStruct((2, _NP, _C), jnp.float32),
        mesh=plsc.VectorSubcoreMesh(**_MESH),
        compiler_params=_SC_PARAMS,
        scratch_types=[
            pltpu.VMEM((18, _SB), jnp.int32),
            pltpu.VMEM((18, _SB), jnp.int32),
            pltpu.VMEM((18, _SB), jnp.int32),
            pltpu.VMEM((18, _SB), jnp.int32),
            pltpu.VMEM((_SB, _C), jnp.float32),
            pltpu.VMEM((_SB, _C), jnp.float32),
            pltpu.VMEM((_SB, _C), jnp.float32),
            pltpu.VMEM((_SB, _C), jnp.float32),
            pltpu.SemaphoreType.DMA,
            pltpu.SemaphoreType.DMA,
            pltpu.SemaphoreType.DMA,
            pltpu.SemaphoreType.DMA,
        ],
    )


def _mm_call(x, w, a, c, kout, leaky, tn=512):
    p = x.shape[0]

    def body(x_ref, w_ref, a_ref, c_ref, o_ref):
        i = pl.program_id(1)
        xv = x_ref[0]
        if leaky:
            xv = jnp.where(xv > 0, xv, 0.01 * xv)
        xv = xv * a_ref[0] + c_ref[0]
        y = lax.dot_general(xv, w_ref[0], (((1,), (0,)), ((), ())),
                            preferred_element_type=jnp.float32)
        rows = i * tn + lax.broadcasted_iota(jnp.int32, (tn, 1), 0)
        o_ref[0] = jnp.where(rows < _N, y, 0.0)

    return pl.pallas_call(
        body,
        grid=(p, _NP // tn),
        in_specs=[
            pl.BlockSpec((1, tn, _C), lambda q, i: (q, i, 0)),
            pl.BlockSpec((1, _C, kout), lambda q, i: (q, 0, 0)),
            pl.BlockSpec((1, 1, _C), lambda q, i: (q, 0, 0)),
            pl.BlockSpec((1, 1, _C), lambda q, i: (q, 0, 0)),
        ],
        out_specs=pl.BlockSpec((1, tn, kout), lambda q, i: (q, i, 0)),
        out_shape=jax.ShapeDtypeStruct((p, _NP, kout), jnp.float32),
    )(x, w, a, c)


def _stats_call(s, tn=2048):
    def body(s_ref, o_ref):
        i = pl.program_id(1)
        x = s_ref[0]
        rows = i * tn + lax.broadcasted_iota(jnp.int32, (tn, 1), 0)
        t = jnp.where(x > 0, x, 0.01 * x)
        t = jnp.where(rows < _N, t, 0.0)
        s1 = jnp.sum(t, axis=0, keepdims=True)
        s2 = jnp.sum(t * t, axis=0, keepdims=True)
        res = jnp.concatenate([s1, s2], axis=0)[None]

        @pl.when(i == 0)
        def _():
            o_ref[...] = res

        @pl.when(i != 0)
        def _():
            o_ref[...] += res

    return pl.pallas_call(
        body,
        grid=(2, _NP // tn),
        in_specs=[pl.BlockSpec((1, tn, _C), lambda q, i: (q, i, 0))],
        out_specs=pl.BlockSpec((1, 2, _C), lambda q, i: (q, 0, 0)),
        out_shape=jax.ShapeDtypeStruct((2, 2, _C), jnp.float32),
    )(s)


def _final_call(s24, ab, tn=1024):
    def body(s_ref, ab_ref, o_ref):
        x2 = s_ref[0]
        x4 = s_ref[1]
        t2 = jnp.where(x2 > 0, x2, 0.01 * x2)
        t4 = jnp.where(x4 > 0, x4, 0.01 * x4)
        o_ref[...] = (t2 * ab_ref[0, 0][None] + ab_ref[0, 1][None]
                      + t4 * ab_ref[1, 0][None] + ab_ref[1, 1][None])

    return pl.pallas_call(
        body,
        grid=(_NP // tn,),
        in_specs=[
            pl.BlockSpec((2, tn, _C), lambda i: (0, i, 0)),
            pl.BlockSpec((2, 2, _C), lambda i: (0, 0, 0)),
        ],
        out_specs=pl.BlockSpec((tn, _C), lambda i: (i, 0)),
        out_shape=jax.ShapeDtypeStruct((_NP, _C), jnp.float32),
    )(s24, ab)


def _fold(st, g, b):
    m = st[0] / _N
    v = st[1] / _N - m * m
    a = g * lax.rsqrt(v + 1e-5)
    return a, b - m * a


def _wcat(w):
    return w.reshape(9, _C, _C).transpose(1, 0, 2).reshape(_C, 9 * _C)


@jax.jit
def kernel(features, coords, W1, W2, W3, W4,
           g0, b0, g02, b02, g1, b1, g2, b2):
    f32 = jnp.float32
    grid = jnp.full((_G, _G, _G), -1, jnp.int32)
    grid = grid.at[coords[:, 0], coords[:, 1], coords[:, 2]].set(
        jnp.arange(features.shape[0], dtype=jnp.int32))
    gridp = jnp.concatenate(
        [grid.reshape(-1), jnp.full((8,), -1, jnp.int32)])

    cpad = jnp.zeros((_NP, 3), jnp.int32).at[:_N].set(coords)
    cz, cy, cx = cpad[:, 0], cpad[:, 1], cpad[:, 2]
    pidx = _idx_call(cz, cy, cx, gridp)

    featp = jnp.zeros((1, _NP, _C), f32).at[0, :_N].set(features)
    w13 = jnp.concatenate([_wcat(W1), _wcat(W3)], axis=1)[None]
    one = jnp.ones((1, 1, _C), f32)
    zero = jnp.zeros((1, 1, _C), f32)
    y13 = _mm_call(featp, w13, one, zero, 18 * _C, leaky=False)
    tab13 = y13.reshape(_NP * 18, _C)

    gsum1 = _make_gsum(18,
                       [(t, t) for t in range(9)],
                       [(9 + t, 9 + t) for t in range(9)])
    s13 = gsum1(tab13, pidx)

    st13 = _stats_call(s13)
    a0, c0 = _fold(st13[0], g0, b0)
    a1, c1 = _fold(st13[1], g1, b1)

    w24 = jnp.stack([_wcat(W2), _wcat(W4)])
    aa = jnp.stack([a0, a1]).reshape(2, 1, _C)
    cc = jnp.stack([c0, c1]).reshape(2, 1, _C)
    y24 = _mm_call(s13, w24, aa, cc, 9 * _C, leaky=True)
    tab24 = y24.reshape(2 * _NP * 9, _C)

    gsum2 = _make_gsum(9,
                       [(9 + j, j) for j in range(9)],
                       [(j, _NP * 9 + j) for j in range(9)])
    s24 = gsum2(tab24, pidx)

    st24 = _stats_call(s24)
    a02, c02 = _fold(st24[0], g02, b02)
    a2, c2 = _fold(st24[1], g2, b2)
    ab = jnp.stack([jnp.stack([a02, c02]), jnp.stack([a2, c2])])
    out = _final_call(s24, ab)
    return out[:_N]


# speedup vs baseline: 1.8025x; 1.8025x over previous
import functools

import jax
import jax.numpy as jnp
from jax import lax
from jax.experimental import pallas as pl
from jax.experimental.pallas import tpu as pltpu
from jax.experimental.pallas import tpu_sc as plsc

_G = 128
_N = 100000
_C = 64
_NW = 32
_BLK = 128
_NBLK = 25
_CHUNK = _BLK * _NBLK
_NP = _NW * _CHUNK
_SENT = _G * _G * _G
_ZP = _N

_OFFS = [(0, d // 3 - 1, d % 3 - 1) for d in range(9)] + \
        [(d // 3 - 1, 0, d % 3 - 1) for d in range(9)]

_MESH = dict(core_axis_name="c", subcore_axis_name="s", num_cores=2,
             num_subcores=16)
_SC_PARAMS = pltpu.CompilerParams(use_tc_tiling_on_sc=False)


def _wid():
    return lax.axis_index("s") * 2 + lax.axis_index("c")


def _idx_body(cz, cy, cx, gridp, pidx, czv, cyv, cxv, linv, gv, outv, sem):
    base = _wid() * _CHUNK
    pltpu.sync_copy(cz.at[pl.ds(base, _CHUNK)], czv)
    pltpu.sync_copy(cy.at[pl.ds(base, _CHUNK)], cyv)
    pltpu.sync_copy(cx.at[pl.ds(base, _CHUNK)], cxv)

    def blk(b, carry):
        b0 = b * _BLK

        def vec(v, c2):
            s = b0 + v * 16
            z = czv[pl.ds(s, 16)]
            y = cyv[pl.ds(s, 16)]
            x = cxv[pl.ds(s, 16)]
            for t, (dz, dy, dx) in enumerate(_OFFS):
                zz = z + dz
                yy = y + dy
                xx = x + dx
                ok = (zz >= 0) & (zz < _G) & (yy >= 0) & (yy < _G) \
                    & (xx >= 0) & (xx < _G)
                lin = (zz * _G + yy) * _G + xx
                linv[t, pl.ds(v * 16, 16)] = jnp.where(ok, lin, _SENT)
            return c2

        lax.fori_loop(0, _BLK // 16, vec, 0)
        descs = [pltpu.async_copy(gridp.at[linv.at[t]], gv.at[t], sem)
                 for t in range(18)]
        for d in descs:
            d.wait()

        def vec2(v, c2):
            sl = pl.ds(v * 16, 16)
            for t in range(18):
                g = gv[t, sl]
                outv[t, sl] = jnp.where(g >= 0, g, _ZP)
            return c2

        lax.fori_loop(0, _BLK // 16, vec2, 0)
        pltpu.sync_copy(outv, pidx.at[:, pl.ds(base + b0, _BLK)])
        return carry

    lax.fori_loop(0, _NBLK, blk, 0)


def _idx_call(cz, cy, cx, gridp):
    return pl.kernel(
        _idx_body,
        out_type=jax.ShapeDtypeStruct((18, _NP), jnp.int32),
        mesh=plsc.VectorSubcoreMesh(**_MESH),
        compiler_params=_SC_PARAMS,
        scratch_types=[
            pltpu.VMEM((_CHUNK,), jnp.int32),
            pltpu.VMEM((_CHUNK,), jnp.int32),
            pltpu.VMEM((_CHUNK,), jnp.int32),
            pltpu.VMEM((18, _BLK), jnp.int32),
            pltpu.VMEM((18, _BLK), jnp.int32),
            pltpu.VMEM((18, _BLK), jnp.int32),
            pltpu.SemaphoreType.DMA,
        ],
    )(cz, cy, cx, gridp)


def _make_gsum(mult, terms0, terms1):
    allt = list(terms0) + list(terms1)

    def body(table, pidx, out, pv, sidx, a0, a1, a2, b0, b1, b2, sem):
        base = _wid() * _CHUNK
        zero16 = jnp.zeros((16,), jnp.float32)
        accs = (a0, a1, a2, b0, b1, b2)

        def blk(b, carry):
            col = base + b * _BLK
            pltpu.sync_copy(pidx.at[:, pl.ds(col, _BLK)], pv)

            def vec(v, c2):
                sl = pl.ds(v * 16, 16)
                for t, (row, addc) in enumerate(allt):
                    sidx[t, sl] = pv[row, sl] * mult + addc
                return c2

            lax.fori_loop(0, _BLK // 16, vec, 0)

            def zrow(r, c2):
                for cc in range(_C // 16):
                    for a in accs:
                        a[r, pl.ds(cc * 16, 16)] = zero16
                return c2

            lax.fori_loop(0, _BLK, zrow, 0)
            descs = [
                pltpu.async_copy(table.at[sidx.at[t]],
                                 accs[(t // 9) * 3 + t % 3], sem, add=True)
                for t in range(18)
            ]
            for d in descs:
                d.wait()

            def srow(r, c2):
                for cc in range(_C // 16):
                    sl = pl.ds(cc * 16, 16)
                    a0[r, sl] = a0[r, sl] + a1[r, sl] + a2[r, sl]
                    b0[r, sl] = b0[r, sl] + b1[r, sl] + b2[r, sl]
                return c2

            lax.fori_loop(0, _BLK, srow, 0)
            pltpu.sync_copy(a0, out.at[0, pl.ds(col, _BLK)])
            pltpu.sync_copy(b0, out.at[1, pl.ds(col, _BLK)])
            return carry

        lax.fori_loop(0, _NBLK, blk, 0)

    return pl.kernel(
        body,
        out_type=jax.ShapeDtypeStruct((2, _NP, _C), jnp.float32),
        mesh=plsc.VectorSubcoreMesh(**_MESH),
        compiler_params=_SC_PARAMS,
        scratch_types=[
            pltpu.VMEM((18, _BLK), jnp.int32),
            pltpu.VMEM((18, _BLK), jnp.int32),
            pltpu.VMEM((_BLK, _C), jnp.float32),
            pltpu.VMEM((_BLK, _C), jnp.float32),
            pltpu.VMEM((_BLK, _C), jnp.float32),
            pltpu.VMEM((_BLK, _C), jnp.float32),
            pltpu.VMEM((_BLK, _C), jnp.float32),
            pltpu.VMEM((_BLK, _C), jnp.float32),
            pltpu.SemaphoreType.DMA,
        ],
    )


def _mm_call(x, w, a, c, kout, leaky, tn=512):
    p = x.shape[0]

    def body(x_ref, w_ref, a_ref, c_ref, o_ref):
        i = pl.program_id(1)
        xv = x_ref[0]
        if leaky:
            xv = jnp.where(xv > 0, xv, 0.01 * xv)
        xv = xv * a_ref[0] + c_ref[0]
        y = lax.dot_general(xv, w_ref[0], (((1,), (0,)), ((), ())),
                            preferred_element_type=jnp.float32)
        rows = i * tn + lax.broadcasted_iota(jnp.int32, (tn, 1), 0)
        o_ref[0] = jnp.where(rows < _N, y, 0.0)

    return pl.pallas_call(
        body,
        grid=(p, _NP // tn),
        in_specs=[
            pl.BlockSpec((1, tn, _C), lambda q, i: (q, i, 0)),
            pl.BlockSpec((1, _C, kout), lambda q, i: (q, 0, 0)),
            pl.BlockSpec((1, 1, _C), lambda q, i: (q, 0, 0)),
            pl.BlockSpec((1, 1, _C), lambda q, i: (q, 0, 0)),
        ],
        out_specs=pl.BlockSpec((1, tn, kout), lambda q, i: (q, i, 0)),
        out_shape=jax.ShapeDtypeStruct((p, _NP, kout), jnp.float32),
    )(x, w, a, c)


def _stats_call(s, tn=2048):
    def body(s_ref, o_ref):
        i = pl.program_id(1)
        x = s_ref[0]
        rows = i * tn + lax.broadcasted_iota(jnp.int32, (tn, 1), 0)
        t = jnp.where(x > 0, x, 0.01 * x)
        t = jnp.where(rows < _N, t, 0.0)
        s1 = jnp.sum(t, axis=0, keepdims=True)
        s2 = jnp.sum(t * t, axis=0, keepdims=True)
        res = jnp.concatenate([s1, s2], axis=0)[None]

        @pl.when(i == 0)
        def _():
            o_ref[...] = res

        @pl.when(i != 0)
        def _():
            o_ref[...] += res

    return pl.pallas_call(
        body,
        grid=(2, _NP // tn),
        in_specs=[pl.BlockSpec((1, tn, _C), lambda q, i: (q, i, 0))],
        out_specs=pl.BlockSpec((1, 2, _C), lambda q, i: (q, 0, 0)),
        out_shape=jax.ShapeDtypeStruct((2, 2, _C), jnp.float32),
    )(s)


def _final_call(s24, ab, tn=1024):
    def body(s_ref, ab_ref, o_ref):
        x2 = s_ref[0]
        x4 = s_ref[1]
        t2 = jnp.where(x2 > 0, x2, 0.01 * x2)
        t4 = jnp.where(x4 > 0, x4, 0.01 * x4)
        o_ref[...] = (t2 * ab_ref[0, 0][None] + ab_ref[0, 1][None]
                      + t4 * ab_ref[1, 0][None] + ab_ref[1, 1][None])

    return pl.pallas_call(
        body,
        grid=(_NP // tn,),
        in_specs=[
            pl.BlockSpec((2, tn, _C), lambda i: (0, i, 0)),
            pl.BlockSpec((2, 2, _C), lambda i: (0, 0, 0)),
        ],
        out_specs=pl.BlockSpec((tn, _C), lambda i: (i, 0)),
        out_shape=jax.ShapeDtypeStruct((_NP, _C), jnp.float32),
    )(s24, ab)


def _fold(st, g, b):
    m = st[0] / _N
    v = st[1] / _N - m * m
    a = g * lax.rsqrt(v + 1e-5)
    return a, b - m * a


def _wcat(w):
    return w.reshape(9, _C, _C).transpose(1, 0, 2).reshape(_C, 9 * _C)


@jax.jit
def kernel(features, coords, W1, W2, W3, W4,
           g0, b0, g02, b02, g1, b1, g2, b2):
    f32 = jnp.float32
    grid = jnp.full((_G, _G, _G), -1, jnp.int32)
    grid = grid.at[coords[:, 0], coords[:, 1], coords[:, 2]].set(
        jnp.arange(features.shape[0], dtype=jnp.int32))
    gridp = jnp.concatenate(
        [grid.reshape(-1), jnp.full((8,), -1, jnp.int32)])

    cpad = jnp.zeros((_NP, 3), jnp.int32).at[:_N].set(coords)
    cz, cy, cx = cpad[:, 0], cpad[:, 1], cpad[:, 2]
    pidx = _idx_call(cz, cy, cx, gridp)

    featp = jnp.zeros((1, _NP, _C), f32).at[0, :_N].set(features)
    w13 = jnp.concatenate([_wcat(W1), _wcat(W3)], axis=1)[None]
    one = jnp.ones((1, 1, _C), f32)
    zero = jnp.zeros((1, 1, _C), f32)
    y13 = _mm_call(featp, w13, one, zero, 18 * _C, leaky=False)
    tab13 = y13.reshape(_NP * 18, _C)

    gsum1 = _make_gsum(18,
                       [(t, t) for t in range(9)],
                       [(9 + t, 9 + t) for t in range(9)])
    s13 = gsum1(tab13, pidx)

    st13 = _stats_call(s13)
    a0, c0 = _fold(st13[0], g0, b0)
    a1, c1 = _fold(st13[1], g1, b1)

    w24 = jnp.stack([_wcat(W2), _wcat(W4)])
    aa = jnp.stack([a0, a1]).reshape(2, 1, _C)
    cc = jnp.stack([c0, c1]).reshape(2, 1, _C)
    y24 = _mm_call(s13, w24, aa, cc, 9 * _C, leaky=True)
    tab24 = y24.reshape(2 * _NP * 9, _C)

    gsum2 = _make_gsum(9,
                       [(9 + j, j) for j in range(9)],
                       [(j, _NP * 9 + j) for j in range(9)])
    s24 = gsum2(tab24, pidx)

    st24 = _stats_call(s24)
    a02, c02 = _fold(st24[0], g02, b02)
    a2, c2 = _fold(st24[1], g2, b2)
    ab = jnp.stack([jnp.stack([a02, c02]), jnp.stack([a2, c2])])
    out = _final_call(s24, ab)
    return out[:_N]


# speedup vs baseline: 1.8096x; 1.0039x over previous
import functools

import jax
import jax.numpy as jnp
from jax import lax
from jax.experimental import pallas as pl
from jax.experimental.pallas import tpu as pltpu
from jax.experimental.pallas import tpu_sc as plsc

_G = 128
_N = 100000
_C = 64
_NW = 32
_BLK = 128
_NBLK = 25
_CHUNK = _BLK * _NBLK
_NP = _NW * _CHUNK
_SENT = _G * _G * _G
_ZP = _N

_OFFS = [(0, d // 3 - 1, d % 3 - 1) for d in range(9)] + \
        [(d // 3 - 1, 0, d % 3 - 1) for d in range(9)]

_MESH = dict(core_axis_name="c", subcore_axis_name="s", num_cores=2,
             num_subcores=16)
_SC_PARAMS = pltpu.CompilerParams(use_tc_tiling_on_sc=False)


def _wid():
    return lax.axis_index("s") * 2 + lax.axis_index("c")


def _idx_body(cz, cy, cx, gridp, pidx, czv, cyv, cxv, linv, gv, outv, sem):
    base = _wid() * _CHUNK
    pltpu.sync_copy(cz.at[pl.ds(base, _CHUNK)], czv)
    pltpu.sync_copy(cy.at[pl.ds(base, _CHUNK)], cyv)
    pltpu.sync_copy(cx.at[pl.ds(base, _CHUNK)], cxv)

    def blk(b, carry):
        b0 = b * _BLK

        def vec(v, c2):
            s = b0 + v * 16
            z = czv[pl.ds(s, 16)]
            y = cyv[pl.ds(s, 16)]
            x = cxv[pl.ds(s, 16)]
            for t, (dz, dy, dx) in enumerate(_OFFS):
                zz = z + dz
                yy = y + dy
                xx = x + dx
                ok = (zz >= 0) & (zz < _G) & (yy >= 0) & (yy < _G) \
                    & (xx >= 0) & (xx < _G)
                lin = (zz * _G + yy) * _G + xx
                linv[t, pl.ds(v * 16, 16)] = jnp.where(ok, lin, _SENT)
            return c2

        lax.fori_loop(0, _BLK // 16, vec, 0)
        descs = [pltpu.async_copy(gridp.at[linv.at[t]], gv.at[t], sem)
                 for t in range(18)]
        for d in descs:
            d.wait()

        def vec2(v, c2):
            sl = pl.ds(v * 16, 16)
            for t in range(18):
                g = gv[t, sl]
                outv[t, sl] = jnp.where(g >= 0, g, _ZP)
            return c2

        lax.fori_loop(0, _BLK // 16, vec2, 0)
        pltpu.sync_copy(outv, pidx.at[:, pl.ds(base + b0, _BLK)])
        return carry

    lax.fori_loop(0, _NBLK, blk, 0)


def _idx_call(cz, cy, cx, gridp):
    return pl.kernel(
        _idx_body,
        out_type=jax.ShapeDtypeStruct((18, _NP), jnp.int32),
        mesh=plsc.VectorSubcoreMesh(**_MESH),
        compiler_params=_SC_PARAMS,
        scratch_types=[
            pltpu.VMEM((_CHUNK,), jnp.int32),
            pltpu.VMEM((_CHUNK,), jnp.int32),
            pltpu.VMEM((_CHUNK,), jnp.int32),
            pltpu.VMEM((18, _BLK), jnp.int32),
            pltpu.VMEM((18, _BLK), jnp.int32),
            pltpu.VMEM((18, _BLK), jnp.int32),
            pltpu.SemaphoreType.DMA,
        ],
    )(cz, cy, cx, gridp)


def _make_gsum(mult, terms0, terms1):
    allt = list(terms0) + list(terms1)

    def body(table, pidx, out, pv, sidx, gbuf, s0, s1, sem):
        base = _wid() * _CHUNK

        def blk(b, carry):
            col = base + b * _BLK
            pltpu.sync_copy(pidx.at[:, pl.ds(col, _BLK)], pv)

            def vec(v, c2):
                sl = pl.ds(v * 16, 16)
                for t, (row, addc) in enumerate(allt):
                    sidx[t, sl] = pv[row, sl] * mult + addc
                return c2

            lax.fori_loop(0, _BLK // 16, vec, 0)
            descs = [
                pltpu.async_copy(table.at[sidx.at[t]], gbuf.at[t], sem)
                for t in range(18)
            ]
            for d in descs:
                d.wait()

            def srow(r, c2):
                for half, dst in ((0, s0), (1, s1)):
                    for g2 in range(2):
                        sl = pl.ds(g2 * 32, 32)
                        v = [gbuf[t, r, sl]
                             for t in range(9 * half, 9 * half + 9)]
                        while len(v) > 1:
                            v = [v[i] + v[i + 1]
                                 for i in range(0, len(v) - 1, 2)] \
                                + ([v[-1]] if len(v) % 2 else [])
                        dst[r, sl] = v[0]
                return c2

            lax.fori_loop(0, _BLK, srow, 0)
            pltpu.sync_copy(s0, out.at[0, pl.ds(col, _BLK)])
            pltpu.sync_copy(s1, out.at[1, pl.ds(col, _BLK)])
            return carry

        lax.fori_loop(0, _NBLK, blk, 0)

    return pl.kernel(
        body,
        out_type=jax.ShapeDtypeStruct((2, _NP, _C), jnp.bfloat16),
        mesh=plsc.VectorSubcoreMesh(**_MESH),
        compiler_params=_SC_PARAMS,
        scratch_types=[
            pltpu.VMEM((18, _BLK), jnp.int32),
            pltpu.VMEM((18, _BLK), jnp.int32),
            pltpu.VMEM((18, _BLK, _C), jnp.bfloat16),
            pltpu.VMEM((_BLK, _C), jnp.bfloat16),
            pltpu.VMEM((_BLK, _C), jnp.bfloat16),
            pltpu.SemaphoreType.DMA,
        ],
    )


def _mm_call(x, w, a, c, kout, leaky, tn=512):
    p = x.shape[0]

    def body(x_ref, w_ref, a_ref, c_ref, o_ref):
        i = pl.program_id(1)
        xv = x_ref[0].astype(jnp.float32)
        if leaky:
            xv = jnp.where(xv > 0, xv, 0.01 * xv)
        xv = xv * a_ref[0] + c_ref[0]
        y = lax.dot_general(xv, w_ref[0], (((1,), (0,)), ((), ())),
                            preferred_element_type=jnp.float32)
        rows = i * tn + lax.broadcasted_iota(jnp.int32, (tn, 1), 0)
        o_ref[0] = jnp.where(rows < _N, y, 0.0).astype(jnp.bfloat16)

    return pl.pallas_call(
        body,
        grid=(p, _NP // tn),
        in_specs=[
            pl.BlockSpec((1, tn, _C), lambda q, i: (q, i, 0)),
            pl.BlockSpec((1, _C, kout), lambda q, i: (q, 0, 0)),
            pl.BlockSpec((1, 1, _C), lambda q, i: (q, 0, 0)),
            pl.BlockSpec((1, 1, _C), lambda q, i: (q, 0, 0)),
        ],
        out_specs=pl.BlockSpec((1, tn, kout), lambda q, i: (q, i, 0)),
        out_shape=jax.ShapeDtypeStruct((p, _NP, kout), jnp.bfloat16),
    )(x, w, a, c)


def _stats_call(s, tn=2048):
    def body(s_ref, o_ref):
        i = pl.program_id(1)
        x = s_ref[0].astype(jnp.float32)
        rows = i * tn + lax.broadcasted_iota(jnp.int32, (tn, 1), 0)
        t = jnp.where(x > 0, x, 0.01 * x)
        t = jnp.where(rows < _N, t, 0.0)
        s1 = jnp.sum(t, axis=0, keepdims=True)
        s2 = jnp.sum(t * t, axis=0, keepdims=True)
        res = jnp.concatenate([s1, s2], axis=0)[None]

        @pl.when(i == 0)
        def _():
            o_ref[...] = res

        @pl.when(i != 0)
        def _():
            o_ref[...] += res

    return pl.pallas_call(
        body,
        grid=(2, _NP // tn),
        in_specs=[pl.BlockSpec((1, tn, _C), lambda q, i: (q, i, 0))],
        out_specs=pl.BlockSpec((1, 2, _C), lambda q, i: (q, 0, 0)),
        out_shape=jax.ShapeDtypeStruct((2, 2, _C), jnp.float32),
    )(s)


def _final_call(s24, ab, tn=1024):
    def body(s_ref, ab_ref, o_ref):
        x2 = s_ref[0].astype(jnp.float32)
        x4 = s_ref[1].astype(jnp.float32)
        t2 = jnp.where(x2 > 0, x2, 0.01 * x2)
        t4 = jnp.where(x4 > 0, x4, 0.01 * x4)
        o_ref[...] = (t2 * ab_ref[0, 0][None] + ab_ref[0, 1][None]
                      + t4 * ab_ref[1, 0][None] + ab_ref[1, 1][None])

    return pl.pallas_call(
        body,
        grid=(_NP // tn,),
        in_specs=[
            pl.BlockSpec((2, tn, _C), lambda i: (0, i, 0)),
            pl.BlockSpec((2, 2, _C), lambda i: (0, 0, 0)),
        ],
        out_specs=pl.BlockSpec((tn, _C), lambda i: (i, 0)),
        out_shape=jax.ShapeDtypeStruct((_NP, _C), jnp.float32),
    )(s24, ab)


def _fold(st, g, b):
    m = st[0] / _N
    v = st[1] / _N - m * m
    a = g * lax.rsqrt(v + 1e-5)
    return a, b - m * a


def _wcat(w):
    return w.reshape(9, _C, _C).transpose(1, 0, 2).reshape(_C, 9 * _C)


@jax.jit
def kernel(features, coords, W1, W2, W3, W4,
           g0, b0, g02, b02, g1, b1, g2, b2):
    f32 = jnp.float32
    grid = jnp.full((_G, _G, _G), -1, jnp.int32)
    grid = grid.at[coords[:, 0], coords[:, 1], coords[:, 2]].set(
        jnp.arange(features.shape[0], dtype=jnp.int32))
    gridp = jnp.concatenate(
        [grid.reshape(-1), jnp.full((8,), -1, jnp.int32)])

    cpad = jnp.zeros((_NP, 3), jnp.int32).at[:_N].set(coords)
    cz, cy, cx = cpad[:, 0], cpad[:, 1], cpad[:, 2]
    pidx = _idx_call(cz, cy, cx, gridp)

    featp = jnp.zeros((1, _NP, _C), f32).at[0, :_N].set(features)
    w13 = jnp.concatenate([_wcat(W1), _wcat(W3)], axis=1)[None]
    one = jnp.ones((1, 1, _C), f32)
    zero = jnp.zeros((1, 1, _C), f32)
    y13 = _mm_call(featp, w13, one, zero, 18 * _C, leaky=False)
    tab13 = y13.reshape(_NP * 18, _C)

    gsum1 = _make_gsum(18,
                       [(t, t) for t in range(9)],
                       [(9 + t, 9 + t) for t in range(9)])
    s13 = gsum1(tab13, pidx)

    st13 = _stats_call(s13)
    a0, c0 = _fold(st13[0], g0, b0)
    a1, c1 = _fold(st13[1], g1, b1)

    w24 = jnp.stack([_wcat(W2), _wcat(W4)])
    aa = jnp.stack([a0, a1]).reshape(2, 1, _C)
    cc = jnp.stack([c0, c1]).reshape(2, 1, _C)
    y24 = _mm_call(s13, w24, aa, cc, 9 * _C, leaky=True)
    tab24 = y24.reshape(2 * _NP * 9, _C)

    gsum2 = _make_gsum(9,
                       [(9 + j, j) for j in range(9)],
                       [(j, _NP * 9 + j) for j in range(9)])
    s24 = gsum2(tab24, pidx)

    st24 = _stats_call(s24)
    a02, c02 = _fold(st24[0], g02, b02)
    a2, c2 = _fold(st24[1], g2, b2)
    ab = jnp.stack([jnp.stack([a02, c02]), jnp.stack([a2, c2])])
    out = _final_call(s24, ab)
    return out[:_N]
